# baseline (speedup 1.0000x reference)
import functools

import jax
import jax.numpy as jnp
from jax.experimental import pallas as pl
from jax.experimental.pallas import tpu as pltpu

_N = 100000
_D = 128
_K = 16
_BLK = 2000
_GRID = _N // _BLK


def _body(gid_ref, sel_ref, x_ref, out_ref, acc_ref):
    i = pl.program_id(0)

    @pl.when(i == 0)
    def _init():
        acc_ref[...] = jnp.zeros_like(acc_ref)

    sub = gid_ref[0] % 8
    rows = jax.lax.broadcasted_iota(jnp.int32, (8, _D), 0)
    sel = jnp.sum(jnp.where(rows == sub, sel_ref[...], 0.0), axis=0,
                  keepdims=True)
    d = x_ref[...] - sel
    sq = d * d
    acc_ref[...] += jnp.sum(sq.reshape(_BLK // 8, 8, _D), axis=0)

    @pl.when(i == _GRID - 1)
    def _finish():
        dist = jnp.sqrt(jnp.sum(acc_ref[...], axis=0, keepdims=True))
        lane = jax.lax.broadcasted_iota(jnp.int32, (1, _D), 1)
        total = jnp.float32(0.0)
        work = dist
        for _ in range(_K):
            m = jnp.min(work)
            total = total + m
            hit = work == m
            first = jnp.min(jnp.where(hit, lane, _D))
            work = jnp.where(lane == first, jnp.float32(jnp.inf), work)
        out_ref[0, 0] = total / _K


@jax.jit
def _run(gid, x):
    grid_spec = pltpu.PrefetchScalarGridSpec(
        num_scalar_prefetch=1,
        grid=(_GRID,),
        in_specs=[
            pl.BlockSpec((8, _D), lambda i, g: (g[0] // 8, 0)),
            pl.BlockSpec((_BLK, _D), lambda i, g: (i, 0)),
        ],
        out_specs=pl.BlockSpec(memory_space=pltpu.SMEM),
        scratch_shapes=[pltpu.VMEM((8, _D), jnp.float32)],
    )
    out = pl.pallas_call(
        _body,
        grid_spec=grid_spec,
        out_shape=jax.ShapeDtypeStruct((1, 1), jnp.float32),
        compiler_params=pltpu.CompilerParams(
            dimension_semantics=("arbitrary",)),
    )(gid, x, x)
    return out[0, 0]


def kernel(group_id, all_latents):
    gid = jnp.asarray(group_id, jnp.int32).reshape(1)
    return _run(gid, all_latents)


# speedup vs baseline: 1.3948x; 1.3948x over previous
import functools

import jax
import jax.numpy as jnp
from jax.experimental import pallas as pl
from jax.experimental.pallas import tpu as pltpu

_N = 100000
_D = 128
_K = 16
_BLK = 4000
_GRID = _N // _BLK


def _tree_sum(a):
    m = a.shape[0]
    while m > 1:
        half = m // 2
        rest = a[2 * half:]
        a = a[:half] + a[half:2 * half]
        if rest.shape[0]:
            a = jnp.concatenate([a, rest], axis=0)
        m = a.shape[0]
    return a[0]


def _body(gid_ref, sel_ref, x_ref, out_ref, acc_ref):
    i = pl.program_id(0)

    @pl.when(i == 0)
    def _init():
        acc_ref[...] = jnp.zeros_like(acc_ref)

    sub = gid_ref[0] % 8
    rows = jax.lax.broadcasted_iota(jnp.int32, (8, _D), 0)
    sel = jnp.sum(jnp.where(rows == sub, sel_ref[...], 0.0), axis=0,
                  keepdims=True)
    d = x_ref[...] - sel
    sq = d * d
    acc_ref[...] += _tree_sum(sq.reshape(_BLK // 8, 8, _D))

    @pl.when(i == _GRID - 1)
    def _finish():
        dist = jnp.sqrt(jnp.sum(acc_ref[...], axis=0, keepdims=True))
        lane = jax.lax.broadcasted_iota(jnp.int32, (1, _D), 1)
        total = jnp.float32(0.0)
        work = dist
        for _ in range(_K):
            m = jnp.min(work)
            total = total + m
            hit = work == m
            first = jnp.min(jnp.where(hit, lane, _D))
            work = jnp.where(lane == first, jnp.float32(jnp.inf), work)
        out_ref[0, 0] = total / _K


@jax.jit
def _run(gid, x):
    grid_spec = pltpu.PrefetchScalarGridSpec(
        num_scalar_prefetch=1,
        grid=(_GRID,),
        in_specs=[
            pl.BlockSpec((8, _D), lambda i, g: (g[0] // 8, 0)),
            pl.BlockSpec((_BLK, _D), lambda i, g: (i, 0)),
        ],
        out_specs=pl.BlockSpec(memory_space=pltpu.SMEM),
        scratch_shapes=[pltpu.VMEM((8, _D), jnp.float32)],
    )
    out = pl.pallas_call(
        _body,
        grid_spec=grid_spec,
        out_shape=jax.ShapeDtypeStruct((1, 1), jnp.float32),
        compiler_params=pltpu.CompilerParams(
            dimension_semantics=("arbitrary",)),
    )(gid, x, x)
    return out[0, 0]


def kernel(group_id, all_latents):
    gid = jnp.asarray(group_id, jnp.int32).reshape(1)
    return _run(gid, all_latents)


# speedup vs baseline: 1.5072x; 1.0806x over previous
import functools

import jax
import jax.numpy as jnp
from jax.experimental import pallas as pl
from jax.experimental.pallas import tpu as pltpu

_N = 100000
_D = 128
_K = 16
_BLK = 4000
_GRID = _N // _BLK
_U = 10
_C = _BLK // (8 * _U)


def _body(gid_ref, sel_ref, x_ref, out_ref, acc_ref):
    i = pl.program_id(0)

    @pl.when(i == 0)
    def _init():
        acc_ref[...] = jnp.zeros_like(acc_ref)

    sub = gid_ref[0] % 8
    rows = jax.lax.broadcasted_iota(jnp.int32, (8, _D), 0)
    sel = jnp.sum(jnp.where(rows == sub, sel_ref[...], 0.0), axis=0,
                  keepdims=True)
    y = x_ref[...].reshape(_U, _C * 8, _D)
    acc = None
    for c in range(_C):
        d = y[:, c * 8:(c + 1) * 8, :] - sel
        s = d * d
        acc = s if acc is None else acc + s
    m = _U
    while m > 1:
        half = m // 2
        rest = acc[2 * half:]
        acc = acc[:half] + acc[half:2 * half]
        if rest.shape[0]:
            acc = jnp.concatenate([acc, rest], axis=0)
        m = acc.shape[0]
    acc_ref[...] += acc[0]

    @pl.when(i == _GRID - 1)
    def _finish():
        dist = jnp.sqrt(jnp.sum(acc_ref[...], axis=0, keepdims=True))
        lane = jax.lax.broadcasted_iota(jnp.int32, (1, _D), 1)
        total = jnp.float32(0.0)
        work = dist
        for _ in range(_K):
            m = jnp.min(work)
            total = total + m
            hit = work == m
            first = jnp.min(jnp.where(hit, lane, _D))
            work = jnp.where(lane == first, jnp.float32(jnp.inf), work)
        out_ref[0, 0] = total / _K


@jax.jit
def _run(gid, x):
    grid_spec = pltpu.PrefetchScalarGridSpec(
        num_scalar_prefetch=1,
        grid=(_GRID,),
        in_specs=[
            pl.BlockSpec((8, _D), lambda i, g: (g[0] // 8, 0)),
            pl.BlockSpec((_BLK, _D), lambda i, g: (i, 0)),
        ],
        out_specs=pl.BlockSpec(memory_space=pltpu.SMEM),
        scratch_shapes=[pltpu.VMEM((8, _D), jnp.float32)],
    )
    out = pl.pallas_call(
        _body,
        grid_spec=grid_spec,
        out_shape=jax.ShapeDtypeStruct((1, 1), jnp.float32),
        compiler_params=pltpu.CompilerParams(
            dimension_semantics=("arbitrary",)),
    )(gid, x, x)
    return out[0, 0]


def kernel(group_id, all_latents):
    gid = jnp.asarray(group_id, jnp.int32).reshape(1)
    return _run(gid, all_latents)


# speedup vs baseline: 1.9707x; 1.3075x over previous
import functools

import jax
import jax.numpy as jnp
from jax.experimental import pallas as pl
from jax.experimental.pallas import tpu as pltpu

_N = 100000
_D = 128
_K = 16
_BLK = 10000
_GRID = _N // _BLK
_U = 25
_C = _BLK // (8 * _U)


def _body(gid_ref, sel_ref, x_ref, out_ref, acc_ref):
    i = pl.program_id(0)

    @pl.when(i == 0)
    def _init():
        acc_ref[...] = jnp.zeros_like(acc_ref)

    sub = gid_ref[0] % 8
    rows = jax.lax.broadcasted_iota(jnp.int32, (8, _D), 0)
    sel = jnp.sum(jnp.where(rows == sub, sel_ref[...], 0.0), axis=0,
                  keepdims=True)
    y = x_ref[...].reshape(_U, _C * 8, _D)
    acc = None
    for c in range(_C):
        d = y[:, c * 8:(c + 1) * 8, :] - sel
        s = d * d
        acc = s if acc is None else acc + s
    m = _U
    while m > 1:
        half = m // 2
        rest = acc[2 * half:]
        acc = acc[:half] + acc[half:2 * half]
        if rest.shape[0]:
            acc = jnp.concatenate([acc, rest], axis=0)
        m = acc.shape[0]
    acc_ref[...] += acc[0]

    @pl.when(i == _GRID - 1)
    def _finish():
        dist = jnp.sqrt(jnp.sum(acc_ref[...], axis=0, keepdims=True))
        lane = jax.lax.broadcasted_iota(jnp.int32, (1, _D), 1)
        total = jnp.float32(0.0)
        work = dist
        for _ in range(_K):
            m = jnp.min(work)
            total = total + m
            hit = work == m
            first = jnp.min(jnp.where(hit, lane, _D))
            work = jnp.where(lane == first, jnp.float32(jnp.inf), work)
        out_ref[0, 0] = total / _K


@jax.jit
def _run(gid, x):
    grid_spec = pltpu.PrefetchScalarGridSpec(
        num_scalar_prefetch=1,
        grid=(_GRID,),
        in_specs=[
            pl.BlockSpec((8, _D), lambda i, g: (g[0] // 8, 0)),
            pl.BlockSpec((_BLK, _D), lambda i, g: (i, 0)),
        ],
        out_specs=pl.BlockSpec(memory_space=pltpu.SMEM),
        scratch_shapes=[pltpu.VMEM((8, _D), jnp.float32)],
    )
    out = pl.pallas_call(
        _body,
        grid_spec=grid_spec,
        out_shape=jax.ShapeDtypeStruct((1, 1), jnp.float32),
        compiler_params=pltpu.CompilerParams(
            dimension_semantics=("arbitrary",)),
    )(gid, x, x)
    return out[0, 0]


def kernel(group_id, all_latents):
    gid = jnp.asarray(group_id, jnp.int32).reshape(1)
    return _run(gid, all_latents)


# speedup vs baseline: 1.9711x; 1.0002x over previous
import functools

import jax
import jax.numpy as jnp
from jax.experimental import pallas as pl
from jax.experimental.pallas import tpu as pltpu

_N = 100000
_D = 128
_K = 16
_BLK = 20000
_GRID = _N // _BLK
_U = 25
_C = _BLK // (8 * _U)


def _body(gid_ref, sel_ref, x_ref, out_ref, acc_ref):
    i = pl.program_id(0)

    @pl.when(i == 0)
    def _init():
        acc_ref[...] = jnp.zeros_like(acc_ref)

    sub = gid_ref[0] % 8
    rows = jax.lax.broadcasted_iota(jnp.int32, (8, _D), 0)
    sel = jnp.sum(jnp.where(rows == sub, sel_ref[...], 0.0), axis=0,
                  keepdims=True)
    y = x_ref[...].reshape(_U, _C * 8, _D)
    acc = None
    for c in range(_C):
        d = y[:, c * 8:(c + 1) * 8, :] - sel
        s = d * d
        acc = s if acc is None else acc + s
    m = _U
    while m > 1:
        half = m // 2
        rest = acc[2 * half:]
        acc = acc[:half] + acc[half:2 * half]
        if rest.shape[0]:
            acc = jnp.concatenate([acc, rest], axis=0)
        m = acc.shape[0]
    acc_ref[...] += acc[0]

    @pl.when(i == _GRID - 1)
    def _finish():
        dist = jnp.sqrt(jnp.sum(acc_ref[...], axis=0, keepdims=True))
        lane = jax.lax.broadcasted_iota(jnp.int32, (1, _D), 1)
        total = jnp.float32(0.0)
        work = dist
        for _ in range(_K):
            m = jnp.min(work)
            total = total + m
            hit = work == m
            first = jnp.min(jnp.where(hit, lane, _D))
            work = jnp.where(lane == first, jnp.float32(jnp.inf), work)
        out_ref[0, 0] = total / _K


@jax.jit
def _run(gid, x):
    grid_spec = pltpu.PrefetchScalarGridSpec(
        num_scalar_prefetch=1,
        grid=(_GRID,),
        in_specs=[
            pl.BlockSpec((8, _D), lambda i, g: (g[0] // 8, 0)),
            pl.BlockSpec((_BLK, _D), lambda i, g: (i, 0)),
        ],
        out_specs=pl.BlockSpec(memory_space=pltpu.SMEM),
        scratch_shapes=[pltpu.VMEM((8, _D), jnp.float32)],
    )
    out = pl.pallas_call(
        _body,
        grid_spec=grid_spec,
        out_shape=jax.ShapeDtypeStruct((1, 1), jnp.float32),
        compiler_params=pltpu.CompilerParams(
            dimension_semantics=("arbitrary",)),
    )(gid, x, x)
    return out[0, 0]


def kernel(group_id, all_latents):
    gid = jnp.asarray(group_id, jnp.int32).reshape(1)
    return _run(gid, all_latents)
